# Initial kernel scaffold; baseline (speedup 1.0000x reference)
import jax
import jax.numpy as jnp
from jax.experimental import pallas as pl


def kernel(x, g, W1, b1, W2, b2):
    raise NotImplementedError("write your pallas kernel here")


# speedup vs baseline: 9.0468x; 9.0468x over previous
import functools

import jax
import jax.numpy as jnp
from jax import lax
from jax.experimental import pallas as pl
from jax.experimental.pallas import tpu as pltpu
from jax.experimental.pallas import tpu_sc as plsc

N_NODES = 10000
N_EDGES = 320000
D = 128

NC = 2
NS = 16
NW = NC * NS

CHUNK = 128
CPW = 80
EPW = CPW * CHUNK
EP = NW * EPW
NP = 10240
ROWS_PER_TILE = NP // NS

_MESH = plsc.VectorSubcoreMesh(
    core_axis_name="c", subcore_axis_name="s", num_cores=NC, num_subcores=NS
)


@functools.partial(
    pl.kernel,
    out_type=jax.ShapeDtypeStruct((NC, NP, 16), jnp.float32),
    mesh=_MESH,
    scratch_types=[
        pltpu.VMEM((CPW, CHUNK), jnp.int32),
        pltpu.VMEM((CHUNK, 16), jnp.float32),
        pltpu.VMEM_SHARED((NP, 16), jnp.float32),
        pltpu.SemaphoreType.DMA,
    ],
)
def _deg_kernel(dst_hbm, ones_hbm, zeros_hbm, out_hbm, dst_v, ones_v, deg_sh, sem):
    c = lax.axis_index("c")
    s = lax.axis_index("s")
    wid = c * NS + s
    pltpu.sync_copy(dst_hbm.at[pl.ds(wid * CPW, CPW)], dst_v)
    pltpu.sync_copy(ones_hbm, ones_v)
    base = s * ROWS_PER_TILE
    pltpu.sync_copy(zeros_hbm.at[pl.ds(base, ROWS_PER_TILE)],
                    deg_sh.at[pl.ds(base, ROWS_PER_TILE)])
    plsc.subcore_barrier()

    def body(j, carry):
        pltpu.sync_copy(ones_v, deg_sh.at[dst_v.at[j]], add=True)
        return carry

    lax.fori_loop(0, CPW, body, None)
    plsc.subcore_barrier()
    pltpu.sync_copy(deg_sh.at[pl.ds(base, ROWS_PER_TILE)],
                    out_hbm.at[c, pl.ds(base, ROWS_PER_TILE)])


@functools.partial(
    pl.kernel,
    out_type=jax.ShapeDtypeStruct((NC, NP, D), jnp.float32),
    mesh=_MESH,
    scratch_types=[
        pltpu.VMEM((CPW, CHUNK), jnp.int32),
        pltpu.VMEM((CPW, CHUNK), jnp.int32),
        pltpu.VMEM((CHUNK, D), jnp.float32),
        pltpu.VMEM_SHARED((NP, D), jnp.float32),
        pltpu.SemaphoreType.DMA,
    ],
)
def _agg_kernel(src_hbm, dst_hbm, h_hbm, zeros_hbm, out_hbm,
                src_v, dst_v, rows_v, agg_sh, sem):
    c = lax.axis_index("c")
    s = lax.axis_index("s")
    wid = c * NS + s
    pltpu.sync_copy(src_hbm.at[pl.ds(wid * CPW, CPW)], src_v)
    pltpu.sync_copy(dst_hbm.at[pl.ds(wid * CPW, CPW)], dst_v)
    base = s * ROWS_PER_TILE
    pltpu.sync_copy(zeros_hbm.at[pl.ds(base, ROWS_PER_TILE)],
                    agg_sh.at[pl.ds(base, ROWS_PER_TILE)])
    plsc.subcore_barrier()

    def body(j, carry):
        pltpu.async_copy(h_hbm.at[src_v.at[j]], rows_v, sem).wait()
        pltpu.sync_copy(rows_v, agg_sh.at[dst_v.at[j]], add=True)
        return carry

    lax.fori_loop(0, CPW, body, None)
    plsc.subcore_barrier()
    pltpu.sync_copy(agg_sh.at[pl.ds(base, ROWS_PER_TILE)],
                    out_hbm.at[c, pl.ds(base, ROWS_PER_TILE)])


_BLK = 2048
_GRID = NP // _BLK

_row_spec = pl.BlockSpec((_BLK, D), lambda i: (i, 0))
_col_spec = pl.BlockSpec((_BLK, 1), lambda i: (i, 0))
_w_spec = pl.BlockSpec((D, D), lambda i: (0, 0))
_b_spec = pl.BlockSpec((1, D), lambda i: (0, 0))


def _mm_scale_body(x_ref, w_ref, d0_ref, d1_ref, o_ref):
    dis = lax.rsqrt(d0_ref[...] + d1_ref[...] + 1.0)
    o_ref[...] = jnp.dot(x_ref[...], w_ref[...],
                         preferred_element_type=jnp.float32) * dis


_mm_scale = pl.pallas_call(
    _mm_scale_body,
    out_shape=jax.ShapeDtypeStruct((NP, D), jnp.float32),
    grid=(_GRID,),
    in_specs=[_row_spec, _w_spec, _col_spec, _col_spec],
    out_specs=_row_spec,
)


def _combine_mm_body(a0_ref, a1_ref, hp_ref, d0_ref, d1_ref, b_ref, w_ref, o_ref):
    dis = lax.rsqrt(d0_ref[...] + d1_ref[...] + 1.0)
    h = (a0_ref[...] + a1_ref[...] + hp_ref[...]) * dis + b_ref[...]
    h = jnp.maximum(h, 0.0)
    o_ref[...] = jnp.dot(h, w_ref[...], preferred_element_type=jnp.float32) * dis


_combine_mm = pl.pallas_call(
    _combine_mm_body,
    out_shape=jax.ShapeDtypeStruct((NP, D), jnp.float32),
    grid=(_GRID,),
    in_specs=[_row_spec, _row_spec, _row_spec, _col_spec, _col_spec, _b_spec,
              _w_spec],
    out_specs=_row_spec,
)


def _final_body(a0_ref, a1_ref, hp_ref, d0_ref, d1_ref, b_ref, o_ref):
    dis = lax.rsqrt(d0_ref[...] + d1_ref[...] + 1.0)
    o_ref[...] = (a0_ref[...] + a1_ref[...] + hp_ref[...]) * dis + b_ref[...]


_final = pl.pallas_call(
    _final_body,
    out_shape=jax.ShapeDtypeStruct((NP, D), jnp.float32),
    grid=(_GRID,),
    in_specs=[_row_spec, _row_spec, _row_spec, _col_spec, _col_spec, _b_spec],
    out_specs=_row_spec,
)


def kernel(x, g, W1, b1, W2, b2):
    src = g[0].astype(jnp.int32)
    dst = g[1].astype(jnp.int32)
    pad = EP - N_EDGES
    src_p = jnp.concatenate([src, jnp.zeros((pad,), jnp.int32)]).reshape(-1, CHUNK)
    dst_p = jnp.concatenate(
        [dst, jnp.full((pad,), N_NODES, jnp.int32)]).reshape(-1, CHUNK)

    x_p = jnp.zeros((NP, D), jnp.float32).at[:N_NODES].set(x)
    zeros128 = jnp.zeros((NP, D), jnp.float32)
    zeros16 = jnp.zeros((NP, 16), jnp.float32)
    ones16 = jnp.ones((CHUNK, 16), jnp.float32)

    deg = _deg_kernel(dst_p, ones16, zeros16)
    d0 = deg[0, :, 0:1]
    d1 = deg[1, :, 0:1]

    h1p = _mm_scale(x_p, W1, d0, d1)
    agg1 = _agg_kernel(src_p, dst_p, h1p, zeros128)
    h2p = _combine_mm(agg1[0], agg1[1], h1p, d0, d1, b1.reshape(1, D), W2)
    agg2 = _agg_kernel(src_p, dst_p, h2p, zeros128)
    out_p = _final(agg2[0], agg2[1], h2p, d0, d1, b2.reshape(1, D))
    return out_p[:N_NODES]


# speedup vs baseline: 9.9192x; 1.0964x over previous
import functools

import jax
import jax.numpy as jnp
from jax import lax
from jax.experimental import pallas as pl
from jax.experimental.pallas import tpu as pltpu
from jax.experimental.pallas import tpu_sc as plsc

N_NODES = 10000
N_EDGES = 320000
D = 128

NC = 2
NS = 16
NW = NC * NS

CHUNK = 128
CPW = 80
GRP = 8
NGPW = CPW // GRP
EPW = CPW * CHUNK
EP = NW * EPW
NP = 10240
ROWS_PER_TILE = NP // NS

_MESH = plsc.VectorSubcoreMesh(
    core_axis_name="c", subcore_axis_name="s", num_cores=NC, num_subcores=NS
)


@functools.partial(
    pl.kernel,
    out_type=jax.ShapeDtypeStruct((NC, NP, 16), jnp.float32),
    mesh=_MESH,
    scratch_types=[
        pltpu.VMEM((CPW, CHUNK), jnp.int32),
        pltpu.VMEM((CHUNK, 16), jnp.float32),
        pltpu.VMEM_SHARED((NP, 16), jnp.float32),
        pltpu.SemaphoreType.DMA,
    ],
)
def _deg_kernel(dst_hbm, ones_hbm, zeros_hbm, out_hbm, dst_v, ones_v, deg_sh, sem):
    c = lax.axis_index("c")
    s = lax.axis_index("s")
    wid = c * NS + s
    pltpu.sync_copy(dst_hbm.at[pl.ds(wid * CPW, CPW)], dst_v)
    pltpu.sync_copy(ones_hbm, ones_v)
    base = s * ROWS_PER_TILE
    pltpu.sync_copy(zeros_hbm.at[pl.ds(base, ROWS_PER_TILE)],
                    deg_sh.at[pl.ds(base, ROWS_PER_TILE)])
    plsc.subcore_barrier()

    def body(j, carry):
        pltpu.sync_copy(ones_v, deg_sh.at[dst_v.at[j]], add=True)
        return carry

    lax.fori_loop(0, CPW, body, None)
    plsc.subcore_barrier()
    pltpu.sync_copy(deg_sh.at[pl.ds(base, ROWS_PER_TILE)],
                    out_hbm.at[c, pl.ds(base, ROWS_PER_TILE)])


@functools.partial(
    pl.kernel,
    out_type=jax.ShapeDtypeStruct((NC, NP, D), jnp.float32),
    mesh=_MESH,
    scratch_types=[
        pltpu.VMEM((CPW, CHUNK), jnp.int32),
        [pltpu.VMEM((CHUNK,), jnp.int32)] * 2,
        [pltpu.VMEM((CHUNK,), jnp.int32)] * 2,
        pltpu.VMEM((2, CHUNK, D), jnp.float32),
        pltpu.VMEM_SHARED((NP, D), jnp.float32),
        [pltpu.SemaphoreType.DMA] * 2,
        [pltpu.SemaphoreType.DMA] * 2,
    ],
)
def _agg_kernel(pk_hbm, h_hbm, zeros_hbm, out_hbm,
                pk_v, src_bufs, dst_bufs, rows_v, agg_sh, gsems, ssems):
    c = lax.axis_index("c")
    s = lax.axis_index("s")
    wid = c * NS + s
    pltpu.sync_copy(pk_hbm.at[pl.ds(wid * CPW, CPW)], pk_v)
    base = s * ROWS_PER_TILE
    pltpu.sync_copy(zeros_hbm.at[pl.ds(base, ROWS_PER_TILE)],
                    agg_sh.at[pl.ds(base, ROWS_PER_TILE)])
    plsc.subcore_barrier()

    def unpack(chunk, p):
        for k in range(CHUNK // 16):
            v = pk_v[chunk, pl.ds(16 * k, 16)]
            src_bufs[p][pl.ds(16 * k, 16)] = v & 0x3FFF
            dst_bufs[p][pl.ds(16 * k, 16)] = v >> 14

    def gather(p):
        return pltpu.async_copy(h_hbm.at[src_bufs[p]], rows_v.at[p], gsems[p])

    def body(j, carry):
        for b in range(2):
            unpack(GRP * j + b, b)
        gathers = {0: gather(0), 1: gather(1)}
        scatters = {}
        for b in range(GRP):
            p = b % 2
            gathers[b].wait()
            scatters[b] = pltpu.async_copy(
                rows_v.at[p], agg_sh.at[dst_bufs[p]], ssems[p], add=True)
            if b + 2 < GRP:
                scatters[b].wait()
                unpack(GRP * j + b + 2, p)
                gathers[b + 2] = gather(p)
        scatters[GRP - 2].wait()
        scatters[GRP - 1].wait()
        return carry

    lax.fori_loop(0, NGPW, body, None)
    plsc.subcore_barrier()
    pltpu.sync_copy(agg_sh.at[pl.ds(base, ROWS_PER_TILE)],
                    out_hbm.at[c, pl.ds(base, ROWS_PER_TILE)])


_BLK = 2048
_GRID = NP // _BLK

_row_spec = pl.BlockSpec((_BLK, D), lambda i: (i, 0))
_col_spec = pl.BlockSpec((_BLK, 1), lambda i: (i, 0))
_w_spec = pl.BlockSpec((D, D), lambda i: (0, 0))
_b_spec = pl.BlockSpec((1, D), lambda i: (0, 0))


def _mm_scale_body(x_ref, w_ref, d0_ref, d1_ref, o_ref):
    dis = lax.rsqrt(d0_ref[...] + d1_ref[...] + 1.0)
    o_ref[...] = jnp.dot(x_ref[...], w_ref[...],
                         preferred_element_type=jnp.float32) * dis


_mm_scale = pl.pallas_call(
    _mm_scale_body,
    out_shape=jax.ShapeDtypeStruct((NP, D), jnp.float32),
    grid=(_GRID,),
    in_specs=[_row_spec, _w_spec, _col_spec, _col_spec],
    out_specs=_row_spec,
)


def _combine_mm_body(a0_ref, a1_ref, hp_ref, d0_ref, d1_ref, b_ref, w_ref, o_ref):
    dis = lax.rsqrt(d0_ref[...] + d1_ref[...] + 1.0)
    h = (a0_ref[...] + a1_ref[...] + hp_ref[...]) * dis + b_ref[...]
    h = jnp.maximum(h, 0.0)
    o_ref[...] = jnp.dot(h, w_ref[...], preferred_element_type=jnp.float32) * dis


_combine_mm = pl.pallas_call(
    _combine_mm_body,
    out_shape=jax.ShapeDtypeStruct((NP, D), jnp.float32),
    grid=(_GRID,),
    in_specs=[_row_spec, _row_spec, _row_spec, _col_spec, _col_spec, _b_spec,
              _w_spec],
    out_specs=_row_spec,
)


def _final_body(a0_ref, a1_ref, hp_ref, d0_ref, d1_ref, b_ref, o_ref):
    dis = lax.rsqrt(d0_ref[...] + d1_ref[...] + 1.0)
    o_ref[...] = (a0_ref[...] + a1_ref[...] + hp_ref[...]) * dis + b_ref[...]


_final = pl.pallas_call(
    _final_body,
    out_shape=jax.ShapeDtypeStruct((NP, D), jnp.float32),
    grid=(_GRID,),
    in_specs=[_row_spec, _row_spec, _row_spec, _col_spec, _col_spec, _b_spec],
    out_specs=_row_spec,
)


def kernel(x, g, W1, b1, W2, b2):
    src = g[0].astype(jnp.int32)
    dst = g[1].astype(jnp.int32)
    pad = EP - N_EDGES
    src_p = jnp.concatenate(
        [src, jnp.zeros((pad,), jnp.int32)]).reshape(-1, CHUNK)
    dst_p = jnp.concatenate(
        [dst, jnp.full((pad,), N_NODES, jnp.int32)]).reshape(-1, CHUNK)
    pk_p = src_p | (dst_p << 14)

    x_p = jnp.zeros((NP, D), jnp.float32).at[:N_NODES].set(x)
    zeros128 = jnp.zeros((NP, D), jnp.float32)
    zeros16 = jnp.zeros((NP, 16), jnp.float32)
    ones16 = jnp.ones((CHUNK, 16), jnp.float32)

    deg = _deg_kernel(dst_p, ones16, zeros16)
    d0 = deg[0, :, 0:1]
    d1 = deg[1, :, 0:1]

    h1p = _mm_scale(x_p, W1, d0, d1)
    agg1 = _agg_kernel(pk_p, h1p, zeros128)
    h2p = _combine_mm(agg1[0], agg1[1], h1p, d0, d1, b1.reshape(1, D), W2)
    agg2 = _agg_kernel(pk_p, h2p, zeros128)
    out_p = _final(agg2[0], agg2[1], h2p, d0, d1, b2.reshape(1, D))
    return out_p[:N_NODES]
